# Initial kernel scaffold; baseline (speedup 1.0000x reference)
import jax
import jax.numpy as jnp
from jax.experimental import pallas as pl


def kernel(x, W_enc, b_enc, W_dec, b_dec):
    raise NotImplementedError("write your pallas kernel here")


# speedup vs baseline: 1.4418x; 1.4418x over previous
import jax
import jax.numpy as jnp
from jax.experimental import pallas as pl
from jax.experimental.pallas import tpu as pltpu

_NEG_INF = float("-inf")


def _enc_kernel(x_ref, w_ref, b_ref, h_ref):
    h_ref[...] = (
        jax.lax.dot_general(
            x_ref[...], w_ref[...],
            (((1,), (1,)), ((), ())),
            preferred_element_type=jnp.float32,
        )
        + b_ref[...]
    )


def _topk_kernel(h_ref, hs_ref, vals_ref, idx_ref, *, k):
    h = h_ref[...]
    rows, nf = h.shape
    iota = jax.lax.broadcasted_iota(jnp.int32, (rows, nf), 1)
    kiota = jax.lax.broadcasted_iota(jnp.int32, (rows, k), 1)

    def body(i, carry):
        hm, vals, idxs = carry
        m = jnp.max(hm, axis=1, keepdims=True)
        ismax = hm == m
        idx = jnp.min(jnp.where(ismax, iota, nf), axis=1, keepdims=True)
        sel = iota == idx
        slot = kiota == i
        vals = jnp.where(slot, m, vals)
        idxs = jnp.where(slot, idx, idxs)
        return jnp.where(sel, _NEG_INF, hm), vals, idxs

    vals0 = jnp.zeros((rows, k), jnp.float32)
    idxs0 = jnp.zeros((rows, k), jnp.int32)
    hm_final, vals, idxs = jax.lax.fori_loop(
        0, k, body, (h, vals0, idxs0), unroll=False)
    vals_ref[...] = vals
    idx_ref[...] = idxs
    mask = hm_final == _NEG_INF
    hs_ref[...] = jnp.where(mask, jnp.maximum(h, 0.0), 0.0)


def _dec_kernel(hs_ref, w_ref, b_ref, o_ref):
    @pl.when(pl.program_id(1) == 0)
    def _():
        o_ref[...] = jnp.broadcast_to(b_ref[...], o_ref.shape)

    o_ref[...] += jax.lax.dot_general(
        hs_ref[...], w_ref[...],
        (((1,), (1,)), ((), ())),
        preferred_element_type=jnp.float32,
    )


def kernel(x, W_enc, b_enc, W_dec, b_dec):
    B, D = x.shape
    NF = W_enc.shape[0]
    K = 64

    b_enc2 = b_enc.reshape(1, NF)
    b_dec2 = b_dec.reshape(1, D)

    R1 = min(256, B)
    F1 = min(2048, NF)
    h = pl.pallas_call(
        _enc_kernel,
        grid=(B // R1, NF // F1),
        in_specs=[
            pl.BlockSpec((R1, D), lambda i, j: (i, 0)),
            pl.BlockSpec((F1, D), lambda i, j: (j, 0)),
            pl.BlockSpec((1, F1), lambda i, j: (0, j)),
        ],
        out_specs=pl.BlockSpec((R1, F1), lambda i, j: (i, j)),
        out_shape=jax.ShapeDtypeStruct((B, NF), jnp.float32),
        compiler_params=pltpu.CompilerParams(
            dimension_semantics=("parallel", "parallel"),
        ),
    )(x, W_enc, b_enc2)

    R2 = min(32, B)
    import functools
    h_sparse, topk_vals, topk_idx = pl.pallas_call(
        functools.partial(_topk_kernel, k=K),
        grid=(B // R2,),
        in_specs=[pl.BlockSpec((R2, NF), lambda i: (i, 0))],
        out_specs=[
            pl.BlockSpec((R2, NF), lambda i: (i, 0)),
            pl.BlockSpec((R2, K), lambda i: (i, 0)),
            pl.BlockSpec((R2, K), lambda i: (i, 0)),
        ],
        out_shape=[
            jax.ShapeDtypeStruct((B, NF), jnp.float32),
            jax.ShapeDtypeStruct((B, K), jnp.float32),
            jax.ShapeDtypeStruct((B, K), jnp.int32),
        ],
        compiler_params=pltpu.CompilerParams(
            dimension_semantics=("parallel",),
        ),
    )(h)
    del topk_vals

    R3 = min(256, B)
    F3 = min(4096, NF)
    x_hat = pl.pallas_call(
        _dec_kernel,
        grid=(B // R3, NF // F3),
        in_specs=[
            pl.BlockSpec((R3, F3), lambda i, j: (i, j)),
            pl.BlockSpec((D, F3), lambda i, j: (0, j)),
            pl.BlockSpec((1, D), lambda i, j: (0, 0)),
        ],
        out_specs=pl.BlockSpec((R3, D), lambda i, j: (i, 0)),
        out_shape=jax.ShapeDtypeStruct((B, D), jnp.float32),
        compiler_params=pltpu.CompilerParams(
            dimension_semantics=("parallel", "arbitrary"),
        ),
    )(h_sparse, W_dec, b_dec2)

    return (x_hat, h_sparse, topk_idx)


# speedup vs baseline: 4.5384x; 3.1476x over previous
import jax
import jax.numpy as jnp
from jax.experimental import pallas as pl
from jax.experimental.pallas import tpu as pltpu

_NEG_INF = float("-inf")


def _enc_kernel(x_ref, w_ref, b_ref, h_ref):
    h_ref[...] = (
        jax.lax.dot_general(
            x_ref[...], w_ref[...],
            (((1,), (1,)), ((), ())),
            preferred_element_type=jnp.float32,
        )
        + b_ref[...]
    )


def _slow_topk(h, iota, k, hs_ref, vals_ref, idx_ref):
    rows, nf = h.shape
    kiota = jax.lax.broadcasted_iota(jnp.int32, (rows, k), 1)

    def body(i, carry):
        hm, vals, idxs = carry
        m = jnp.max(hm, axis=1, keepdims=True)
        ismax = hm == m
        idx = jnp.min(jnp.where(ismax, iota, nf), axis=1, keepdims=True)
        sel = iota == idx
        slot = kiota == i
        vals = jnp.where(slot, m, vals)
        idxs = jnp.where(slot, idx, idxs)
        return jnp.where(sel, _NEG_INF, hm), vals, idxs

    vals0 = jnp.zeros((rows, k), jnp.float32)
    idxs0 = jnp.zeros((rows, k), jnp.int32)
    hm_final, vals, idxs = jax.lax.fori_loop(
        0, k, body, (h, vals0, idxs0), unroll=False)
    vals_ref[...] = vals
    idx_ref[...] = idxs
    mask = hm_final == _NEG_INF
    hs_ref[...] = jnp.where(mask, jnp.maximum(h, 0.0), 0.0)


def _topk_kernel(h_ref, hs_ref, vals_ref, idx_ref, *, k, s=6, chunk=128):
    h = h_ref[...]
    rows, nf = h.shape
    nchunks = nf // chunk
    h3 = h.reshape(rows, nchunks, chunk)
    lio = jax.lax.broadcasted_iota(jnp.int32, (rows, nchunks, chunk), 2)
    cio2 = jax.lax.broadcasted_iota(jnp.int32, (rows, nchunks), 1)
    iota = jax.lax.broadcasted_iota(jnp.int32, (rows, nf), 1)

    cand_v = []
    cand_i = []
    h3m = h3
    for _ in range(s):
        m = jnp.max(h3m, axis=2, keepdims=True)
        ismax = h3m == m
        lidx = jnp.min(jnp.where(ismax, lio, chunk), axis=2, keepdims=True)
        sel = lio == lidx
        h3m = jnp.where(sel, _NEG_INF, h3m)
        cand_v.append(m.reshape(rows, nchunks))
        cand_i.append(cio2 * chunk + lidx.reshape(rows, nchunks))
    v_last = cand_v[-1]
    cv = jnp.concatenate(cand_v, axis=1)
    ci = jnp.concatenate(cand_i, axis=1)

    kiota = jax.lax.broadcasted_iota(jnp.int32, (rows, k), 1)

    def mbody(i, carry):
        cvm, vals, idxs = carry
        m = jnp.max(cvm, axis=1, keepdims=True)
        ism = cvm == m
        oidx = jnp.min(jnp.where(ism, ci, nf), axis=1, keepdims=True)
        selc = ism & (ci == oidx)
        slot = kiota == i
        vals = jnp.where(slot, m, vals)
        idxs = jnp.where(slot, oidx, idxs)
        return jnp.where(selc, _NEG_INF, cvm), vals, idxs

    vals0 = jnp.zeros((rows, k), jnp.float32)
    idxs0 = jnp.zeros((rows, k), jnp.int32)
    _, vals, idxs = jax.lax.fori_loop(
        0, k, mbody, (cv, vals0, idxs0), unroll=False)

    t_hat = jnp.min(vals, axis=1, keepdims=True)
    viol_cert = jnp.max(v_last, axis=1, keepdims=True) >= t_hat
    ge = (h >= t_hat).astype(jnp.float32)
    cnt = jnp.sum(ge, axis=1, keepdims=True)
    bad_rows = jnp.logical_or(viol_cert, cnt != float(k))
    bad = jnp.max(bad_rows.astype(jnp.int32)) > 0

    @pl.when(jnp.logical_not(bad))
    def _fast():
        vals_ref[...] = vals
        idx_ref[...] = idxs
        hs_ref[...] = jnp.where(h >= t_hat, jnp.maximum(h, 0.0), 0.0)

    @pl.when(bad)
    def _slow():
        _slow_topk(h, iota, k, hs_ref, vals_ref, idx_ref)


def _dec_kernel(hs_ref, w_ref, b_ref, o_ref):
    @pl.when(pl.program_id(1) == 0)
    def _():
        o_ref[...] = jnp.broadcast_to(b_ref[...], o_ref.shape)

    o_ref[...] += jax.lax.dot_general(
        hs_ref[...], w_ref[...],
        (((1,), (1,)), ((), ())),
        preferred_element_type=jnp.float32,
    )


def kernel(x, W_enc, b_enc, W_dec, b_dec):
    B, D = x.shape
    NF = W_enc.shape[0]
    K = 64

    b_enc2 = b_enc.reshape(1, NF)
    b_dec2 = b_dec.reshape(1, D)

    R1 = min(256, B)
    F1 = min(2048, NF)
    h = pl.pallas_call(
        _enc_kernel,
        grid=(B // R1, NF // F1),
        in_specs=[
            pl.BlockSpec((R1, D), lambda i, j: (i, 0)),
            pl.BlockSpec((F1, D), lambda i, j: (j, 0)),
            pl.BlockSpec((1, F1), lambda i, j: (0, j)),
        ],
        out_specs=pl.BlockSpec((R1, F1), lambda i, j: (i, j)),
        out_shape=jax.ShapeDtypeStruct((B, NF), jnp.float32),
        compiler_params=pltpu.CompilerParams(
            dimension_semantics=("parallel", "parallel"),
        ),
    )(x, W_enc, b_enc2)

    R2 = min(32, B)
    import functools
    h_sparse, topk_vals, topk_idx = pl.pallas_call(
        functools.partial(_topk_kernel, k=K),
        grid=(B // R2,),
        in_specs=[pl.BlockSpec((R2, NF), lambda i: (i, 0))],
        out_specs=[
            pl.BlockSpec((R2, NF), lambda i: (i, 0)),
            pl.BlockSpec((R2, K), lambda i: (i, 0)),
            pl.BlockSpec((R2, K), lambda i: (i, 0)),
        ],
        out_shape=[
            jax.ShapeDtypeStruct((B, NF), jnp.float32),
            jax.ShapeDtypeStruct((B, K), jnp.float32),
            jax.ShapeDtypeStruct((B, K), jnp.int32),
        ],
        compiler_params=pltpu.CompilerParams(
            dimension_semantics=("parallel",),
        ),
    )(h)
    del topk_vals

    R3 = min(256, B)
    F3 = min(4096, NF)
    x_hat = pl.pallas_call(
        _dec_kernel,
        grid=(B // R3, NF // F3),
        in_specs=[
            pl.BlockSpec((R3, F3), lambda i, j: (i, j)),
            pl.BlockSpec((D, F3), lambda i, j: (0, j)),
            pl.BlockSpec((1, D), lambda i, j: (0, 0)),
        ],
        out_specs=pl.BlockSpec((R3, D), lambda i, j: (i, 0)),
        out_shape=jax.ShapeDtypeStruct((B, D), jnp.float32),
        compiler_params=pltpu.CompilerParams(
            dimension_semantics=("parallel", "arbitrary"),
        ),
    )(h_sparse, W_dec, b_dec2)

    return (x_hat, h_sparse, topk_idx)


# speedup vs baseline: 5.2428x; 1.1552x over previous
import jax
import jax.numpy as jnp
from jax.experimental import pallas as pl
from jax.experimental.pallas import tpu as pltpu

_NEG_INF = float("-inf")


def _enc_kernel(x_ref, w_ref, b_ref, h_ref):
    h_ref[...] = (
        jax.lax.dot_general(
            x_ref[...], w_ref[...],
            (((1,), (1,)), ((), ())),
            preferred_element_type=jnp.float32,
        )
        + b_ref[...]
    )


def _slow_topk(h, iota, k, hs_ref, vals_ref, idx_ref):
    rows, nf = h.shape
    kiota = jax.lax.broadcasted_iota(jnp.int32, (rows, k), 1)

    def body(i, carry):
        hm, vals, idxs = carry
        m = jnp.max(hm, axis=1, keepdims=True)
        ismax = hm == m
        idx = jnp.min(jnp.where(ismax, iota, nf), axis=1, keepdims=True)
        sel = iota == idx
        slot = kiota == i
        vals = jnp.where(slot, m, vals)
        idxs = jnp.where(slot, idx, idxs)
        return jnp.where(sel, _NEG_INF, hm), vals, idxs

    vals0 = jnp.zeros((rows, k), jnp.float32)
    idxs0 = jnp.zeros((rows, k), jnp.int32)
    hm_final, vals, idxs = jax.lax.fori_loop(
        0, k, body, (h, vals0, idxs0), unroll=False)
    vals_ref[...] = vals
    idx_ref[...] = idxs
    mask = hm_final == _NEG_INF
    hs_ref[...] = jnp.where(mask, jnp.maximum(h, 0.0), 0.0)


def _topk_kernel(h_ref, hs_ref, vals_ref, idx_ref, *, k, s=6, chunk=128):
    h = h_ref[...]
    rows, nf = h.shape
    nchunks = nf // chunk
    h3 = h.reshape(rows, nchunks, chunk)
    lio = jax.lax.broadcasted_iota(jnp.int32, (rows, nchunks, chunk), 2)
    cio2 = jax.lax.broadcasted_iota(jnp.int32, (rows, nchunks), 1)
    iota = jax.lax.broadcasted_iota(jnp.int32, (rows, nf), 1)

    cand_v = []
    cand_i = []
    h3m = h3
    for _ in range(s):
        m = jnp.max(h3m, axis=2, keepdims=True)
        lidx = jnp.argmax(h3m, axis=2).astype(jnp.int32)[:, :, None]
        sel = lio == lidx
        h3m = jnp.where(sel, _NEG_INF, h3m)
        cand_v.append(m.reshape(rows, nchunks))
        cand_i.append(cio2 * chunk + lidx.reshape(rows, nchunks))
    v_last = cand_v[-1]
    cv = jnp.concatenate(cand_v, axis=1)
    ci = jnp.concatenate(cand_i, axis=1)

    kiota = jax.lax.broadcasted_iota(jnp.int32, (rows, k), 1)

    def mbody(i, carry):
        cvm, vals, idxs = carry
        m = jnp.max(cvm, axis=1, keepdims=True)
        ism = cvm == m
        oidx = jnp.min(jnp.where(ism, ci, nf), axis=1, keepdims=True)
        selc = ism & (ci == oidx)
        slot = kiota == i
        vals = jnp.where(slot, m, vals)
        idxs = jnp.where(slot, oidx, idxs)
        return jnp.where(selc, _NEG_INF, cvm), vals, idxs

    vals0 = jnp.zeros((rows, k), jnp.float32)
    idxs0 = jnp.zeros((rows, k), jnp.int32)
    _, vals, idxs = jax.lax.fori_loop(
        0, k, mbody, (cv, vals0, idxs0), unroll=False)

    t_hat = jnp.min(vals, axis=1, keepdims=True)
    viol_cert = jnp.max(v_last, axis=1, keepdims=True) >= t_hat
    ge = (h >= t_hat).astype(jnp.float32)
    cnt = jnp.sum(ge, axis=1, keepdims=True)
    bad_rows = jnp.logical_or(viol_cert, cnt != float(k))
    bad = jnp.max(bad_rows.astype(jnp.int32)) > 0

    @pl.when(jnp.logical_not(bad))
    def _fast():
        vals_ref[...] = vals
        idx_ref[...] = idxs
        hs_ref[...] = jnp.where(h >= t_hat, jnp.maximum(h, 0.0), 0.0)

    @pl.when(bad)
    def _slow():
        _slow_topk(h, iota, k, hs_ref, vals_ref, idx_ref)


def _dec_kernel(hs_ref, w_ref, b_ref, o_ref):
    @pl.when(pl.program_id(1) == 0)
    def _():
        o_ref[...] = jnp.broadcast_to(b_ref[...], o_ref.shape)

    o_ref[...] += jax.lax.dot_general(
        hs_ref[...], w_ref[...],
        (((1,), (1,)), ((), ())),
        preferred_element_type=jnp.float32,
    )


def kernel(x, W_enc, b_enc, W_dec, b_dec):
    B, D = x.shape
    NF = W_enc.shape[0]
    K = 64

    b_enc2 = b_enc.reshape(1, NF)
    b_dec2 = b_dec.reshape(1, D)

    R1 = min(256, B)
    F1 = min(2048, NF)
    h = pl.pallas_call(
        _enc_kernel,
        grid=(B // R1, NF // F1),
        in_specs=[
            pl.BlockSpec((R1, D), lambda i, j: (i, 0)),
            pl.BlockSpec((F1, D), lambda i, j: (j, 0)),
            pl.BlockSpec((1, F1), lambda i, j: (0, j)),
        ],
        out_specs=pl.BlockSpec((R1, F1), lambda i, j: (i, j)),
        out_shape=jax.ShapeDtypeStruct((B, NF), jnp.float32),
        compiler_params=pltpu.CompilerParams(
            dimension_semantics=("parallel", "parallel"),
        ),
    )(x, W_enc, b_enc2)

    R2 = min(32, B)
    import functools
    h_sparse, topk_vals, topk_idx = pl.pallas_call(
        functools.partial(_topk_kernel, k=K),
        grid=(B // R2,),
        in_specs=[pl.BlockSpec((R2, NF), lambda i: (i, 0))],
        out_specs=[
            pl.BlockSpec((R2, NF), lambda i: (i, 0)),
            pl.BlockSpec((R2, K), lambda i: (i, 0)),
            pl.BlockSpec((R2, K), lambda i: (i, 0)),
        ],
        out_shape=[
            jax.ShapeDtypeStruct((B, NF), jnp.float32),
            jax.ShapeDtypeStruct((B, K), jnp.float32),
            jax.ShapeDtypeStruct((B, K), jnp.int32),
        ],
        compiler_params=pltpu.CompilerParams(
            dimension_semantics=("parallel",),
        ),
    )(h)
    del topk_vals

    R3 = min(256, B)
    F3 = min(4096, NF)
    x_hat = pl.pallas_call(
        _dec_kernel,
        grid=(B // R3, NF // F3),
        in_specs=[
            pl.BlockSpec((R3, F3), lambda i, j: (i, j)),
            pl.BlockSpec((D, F3), lambda i, j: (0, j)),
            pl.BlockSpec((1, D), lambda i, j: (0, 0)),
        ],
        out_specs=pl.BlockSpec((R3, D), lambda i, j: (i, 0)),
        out_shape=jax.ShapeDtypeStruct((B, D), jnp.float32),
        compiler_params=pltpu.CompilerParams(
            dimension_semantics=("parallel", "arbitrary"),
        ),
    )(h_sparse, W_dec, b_dec2)

    return (x_hat, h_sparse, topk_idx)


# speedup vs baseline: 5.4818x; 1.0456x over previous
import jax
import jax.numpy as jnp
from jax.experimental import pallas as pl
from jax.experimental.pallas import tpu as pltpu

_NEG_INF = float("-inf")


def _enc_kernel(x_ref, w_ref, b_ref, h_ref):
    h_ref[...] = (
        jax.lax.dot_general(
            x_ref[...], w_ref[...],
            (((1,), (1,)), ((), ())),
            preferred_element_type=jnp.float32,
        )
        + b_ref[...]
    )


def _slow_topk(h, iota, k, hs_ref, vals_ref, idx_ref):
    rows, nf = h.shape
    kiota = jax.lax.broadcasted_iota(jnp.int32, (rows, k), 1)

    def body(i, carry):
        hm, vals, idxs = carry
        m = jnp.max(hm, axis=1, keepdims=True)
        ismax = hm == m
        idx = jnp.min(jnp.where(ismax, iota, nf), axis=1, keepdims=True)
        sel = iota == idx
        slot = kiota == i
        vals = jnp.where(slot, m, vals)
        idxs = jnp.where(slot, idx, idxs)
        return jnp.where(sel, _NEG_INF, hm), vals, idxs

    vals0 = jnp.zeros((rows, k), jnp.float32)
    idxs0 = jnp.zeros((rows, k), jnp.int32)
    hm_final, vals, idxs = jax.lax.fori_loop(
        0, k, body, (h, vals0, idxs0), unroll=False)
    vals_ref[...] = vals
    idx_ref[...] = idxs
    mask = hm_final == _NEG_INF
    hs_ref[...] = jnp.where(mask, jnp.maximum(h, 0.0), 0.0)


def _topk_kernel(h_ref, hs_ref, vals_ref, idx_ref, *, k, s=6, chunk=128):
    h = h_ref[...]
    rows, nf = h.shape
    nchunks = nf // chunk
    h3 = h.reshape(rows, nchunks, chunk)
    lio = jax.lax.broadcasted_iota(jnp.int32, (rows, nchunks, chunk), 2)
    cio2 = jax.lax.broadcasted_iota(jnp.int32, (rows, nchunks), 1)
    iota = jax.lax.broadcasted_iota(jnp.int32, (rows, nf), 1)

    cand_v = []
    cand_i = []
    h3m = h3
    for _ in range(s):
        m = jnp.max(h3m, axis=2, keepdims=True)
        lidx = jnp.argmax(h3m, axis=2).astype(jnp.int32)[:, :, None]
        sel = lio == lidx
        h3m = jnp.where(sel, _NEG_INF, h3m)
        cand_v.append(m.reshape(rows, nchunks))
        cand_i.append(cio2 * chunk + lidx.reshape(rows, nchunks))
    v_last = cand_v[-1]
    cv = jnp.concatenate(cand_v, axis=1)
    ci = jnp.concatenate(cand_i, axis=1)

    kiota = jax.lax.broadcasted_iota(jnp.int32, (rows, k), 1)

    def mbody(i, carry):
        cvm, vals, idxs = carry
        m = jnp.max(cvm, axis=1, keepdims=True)
        ism = cvm == m
        oidx = jnp.min(jnp.where(ism, ci, nf), axis=1, keepdims=True)
        selc = ism & (ci == oidx)
        slot = kiota == i
        vals = jnp.where(slot, m, vals)
        idxs = jnp.where(slot, oidx, idxs)
        return jnp.where(selc, _NEG_INF, cvm), vals, idxs

    vals0 = jnp.zeros((rows, k), jnp.float32)
    idxs0 = jnp.zeros((rows, k), jnp.int32)
    _, vals, idxs = jax.lax.fori_loop(
        0, k, mbody, (cv, vals0, idxs0), unroll=4)

    t_hat = jnp.min(vals, axis=1, keepdims=True)
    viol_cert = jnp.max(v_last, axis=1, keepdims=True) >= t_hat
    ge = (h >= t_hat).astype(jnp.float32)
    cnt = jnp.sum(ge, axis=1, keepdims=True)
    bad_rows = jnp.logical_or(viol_cert, cnt != float(k))
    bad = jnp.max(bad_rows.astype(jnp.int32)) > 0

    @pl.when(jnp.logical_not(bad))
    def _fast():
        vals_ref[...] = vals
        idx_ref[...] = idxs
        hs_ref[...] = jnp.where(h >= t_hat, jnp.maximum(h, 0.0), 0.0)

    @pl.when(bad)
    def _slow():
        _slow_topk(h, iota, k, hs_ref, vals_ref, idx_ref)


def _dec_kernel(hs_ref, w_ref, b_ref, o_ref):
    @pl.when(pl.program_id(1) == 0)
    def _():
        o_ref[...] = jnp.broadcast_to(b_ref[...], o_ref.shape)

    o_ref[...] += jax.lax.dot_general(
        hs_ref[...], w_ref[...],
        (((1,), (1,)), ((), ())),
        preferred_element_type=jnp.float32,
    )


def kernel(x, W_enc, b_enc, W_dec, b_dec):
    B, D = x.shape
    NF = W_enc.shape[0]
    K = 64

    b_enc2 = b_enc.reshape(1, NF)
    b_dec2 = b_dec.reshape(1, D)

    R1 = min(256, B)
    F1 = min(2048, NF)
    h = pl.pallas_call(
        _enc_kernel,
        grid=(B // R1, NF // F1),
        in_specs=[
            pl.BlockSpec((R1, D), lambda i, j: (i, 0)),
            pl.BlockSpec((F1, D), lambda i, j: (j, 0)),
            pl.BlockSpec((1, F1), lambda i, j: (0, j)),
        ],
        out_specs=pl.BlockSpec((R1, F1), lambda i, j: (i, j)),
        out_shape=jax.ShapeDtypeStruct((B, NF), jnp.float32),
        compiler_params=pltpu.CompilerParams(
            dimension_semantics=("parallel", "parallel"),
        ),
    )(x, W_enc, b_enc2)

    R2 = min(32, B)
    import functools
    h_sparse, topk_vals, topk_idx = pl.pallas_call(
        functools.partial(_topk_kernel, k=K),
        grid=(B // R2,),
        in_specs=[pl.BlockSpec((R2, NF), lambda i: (i, 0))],
        out_specs=[
            pl.BlockSpec((R2, NF), lambda i: (i, 0)),
            pl.BlockSpec((R2, K), lambda i: (i, 0)),
            pl.BlockSpec((R2, K), lambda i: (i, 0)),
        ],
        out_shape=[
            jax.ShapeDtypeStruct((B, NF), jnp.float32),
            jax.ShapeDtypeStruct((B, K), jnp.float32),
            jax.ShapeDtypeStruct((B, K), jnp.int32),
        ],
        compiler_params=pltpu.CompilerParams(
            dimension_semantics=("parallel",),
        ),
    )(h)
    del topk_vals

    R3 = min(256, B)
    F3 = min(4096, NF)
    x_hat = pl.pallas_call(
        _dec_kernel,
        grid=(B // R3, NF // F3),
        in_specs=[
            pl.BlockSpec((R3, F3), lambda i, j: (i, j)),
            pl.BlockSpec((D, F3), lambda i, j: (0, j)),
            pl.BlockSpec((1, D), lambda i, j: (0, 0)),
        ],
        out_specs=pl.BlockSpec((R3, D), lambda i, j: (i, 0)),
        out_shape=jax.ShapeDtypeStruct((B, D), jnp.float32),
        compiler_params=pltpu.CompilerParams(
            dimension_semantics=("parallel", "arbitrary"),
        ),
    )(h_sparse, W_dec, b_dec2)

    return (x_hat, h_sparse, topk_idx)


# speedup vs baseline: 5.9335x; 1.0824x over previous
import jax
import jax.numpy as jnp
from jax.experimental import pallas as pl
from jax.experimental.pallas import tpu as pltpu

_NEG_INF = float("-inf")


def _enc_kernel(x_ref, w_ref, b_ref, h_ref):
    h_ref[...] = (
        jax.lax.dot_general(
            x_ref[...], w_ref[...],
            (((1,), (1,)), ((), ())),
            preferred_element_type=jnp.float32,
        )
        + b_ref[...]
    )


def _slow_topk(h, iota, k, hs_ref, vals_ref, idx_ref):
    rows, nf = h.shape
    kiota = jax.lax.broadcasted_iota(jnp.int32, (rows, k), 1)

    def body(i, carry):
        hm, vals, idxs = carry
        m = jnp.max(hm, axis=1, keepdims=True)
        ismax = hm == m
        idx = jnp.min(jnp.where(ismax, iota, nf), axis=1, keepdims=True)
        sel = iota == idx
        slot = kiota == i
        vals = jnp.where(slot, m, vals)
        idxs = jnp.where(slot, idx, idxs)
        return jnp.where(sel, _NEG_INF, hm), vals, idxs

    vals0 = jnp.zeros((rows, k), jnp.float32)
    idxs0 = jnp.zeros((rows, k), jnp.int32)
    hm_final, vals, idxs = jax.lax.fori_loop(
        0, k, body, (h, vals0, idxs0), unroll=False)
    vals_ref[...] = vals
    idx_ref[...] = idxs
    mask = hm_final == _NEG_INF
    hs_ref[...] = jnp.where(mask, jnp.maximum(h, 0.0), 0.0)


def _topk_kernel(h_ref, hs_ref, vals_ref, idx_ref, *, k, s=6, chunk=128):
    h = h_ref[...]
    rows, nf = h.shape
    nper = nf // chunk
    h3 = h.reshape(rows, nper, chunk)
    jio = jax.lax.broadcasted_iota(jnp.int32, (rows, nper, chunk), 1)
    lane2 = jax.lax.broadcasted_iota(jnp.int32, (rows, chunk), 1)
    iota = jax.lax.broadcasted_iota(jnp.int32, (rows, nf), 1)

    cand_v = []
    cand_i = []
    h3m = h3
    for _ in range(s):
        m = jnp.max(h3m, axis=1)
        jidx = jnp.argmax(h3m, axis=1).astype(jnp.int32)
        sel = jio == jidx[:, None, :]
        h3m = jnp.where(sel, _NEG_INF, h3m)
        cand_v.append(m)
        cand_i.append(jidx * chunk + lane2)
    v_last = cand_v[-1]
    cv = jnp.concatenate(cand_v, axis=1)
    ci = jnp.concatenate(cand_i, axis=1)

    kiota = jax.lax.broadcasted_iota(jnp.int32, (rows, k), 1)

    def mbody(i, carry):
        cvm, vals, idxs = carry
        m = jnp.max(cvm, axis=1, keepdims=True)
        ism = cvm == m
        oidx = jnp.min(jnp.where(ism, ci, nf), axis=1, keepdims=True)
        selc = ism & (ci == oidx)
        slot = kiota == i
        vals = jnp.where(slot, m, vals)
        idxs = jnp.where(slot, oidx, idxs)
        return jnp.where(selc, _NEG_INF, cvm), vals, idxs

    vals0 = jnp.zeros((rows, k), jnp.float32)
    idxs0 = jnp.zeros((rows, k), jnp.int32)
    _, vals, idxs = jax.lax.fori_loop(
        0, k, mbody, (cv, vals0, idxs0), unroll=4)

    t_hat = jnp.min(vals, axis=1, keepdims=True)
    viol_cert = jnp.max(v_last, axis=1, keepdims=True) >= t_hat
    ge = (h >= t_hat).astype(jnp.float32)
    cnt = jnp.sum(ge, axis=1, keepdims=True)
    bad_rows = jnp.logical_or(viol_cert, cnt != float(k))
    bad = jnp.max(bad_rows.astype(jnp.int32)) > 0

    @pl.when(jnp.logical_not(bad))
    def _fast():
        vals_ref[...] = vals
        idx_ref[...] = idxs
        hs_ref[...] = jnp.where(h >= t_hat, jnp.maximum(h, 0.0), 0.0)

    @pl.when(bad)
    def _slow():
        _slow_topk(h, iota, k, hs_ref, vals_ref, idx_ref)


def _dec_kernel(hs_ref, w_ref, b_ref, o_ref):
    @pl.when(pl.program_id(1) == 0)
    def _():
        o_ref[...] = jnp.broadcast_to(b_ref[...], o_ref.shape)

    o_ref[...] += jax.lax.dot_general(
        hs_ref[...], w_ref[...],
        (((1,), (1,)), ((), ())),
        preferred_element_type=jnp.float32,
    )


def kernel(x, W_enc, b_enc, W_dec, b_dec):
    B, D = x.shape
    NF = W_enc.shape[0]
    K = 64

    b_enc2 = b_enc.reshape(1, NF)
    b_dec2 = b_dec.reshape(1, D)

    R1 = min(256, B)
    F1 = min(2048, NF)
    h = pl.pallas_call(
        _enc_kernel,
        grid=(B // R1, NF // F1),
        in_specs=[
            pl.BlockSpec((R1, D), lambda i, j: (i, 0)),
            pl.BlockSpec((F1, D), lambda i, j: (j, 0)),
            pl.BlockSpec((1, F1), lambda i, j: (0, j)),
        ],
        out_specs=pl.BlockSpec((R1, F1), lambda i, j: (i, j)),
        out_shape=jax.ShapeDtypeStruct((B, NF), jnp.float32),
        compiler_params=pltpu.CompilerParams(
            dimension_semantics=("parallel", "parallel"),
        ),
    )(x, W_enc, b_enc2)

    R2 = min(32, B)
    import functools
    h_sparse, topk_vals, topk_idx = pl.pallas_call(
        functools.partial(_topk_kernel, k=K),
        grid=(B // R2,),
        in_specs=[pl.BlockSpec((R2, NF), lambda i: (i, 0))],
        out_specs=[
            pl.BlockSpec((R2, NF), lambda i: (i, 0)),
            pl.BlockSpec((R2, K), lambda i: (i, 0)),
            pl.BlockSpec((R2, K), lambda i: (i, 0)),
        ],
        out_shape=[
            jax.ShapeDtypeStruct((B, NF), jnp.float32),
            jax.ShapeDtypeStruct((B, K), jnp.float32),
            jax.ShapeDtypeStruct((B, K), jnp.int32),
        ],
        compiler_params=pltpu.CompilerParams(
            dimension_semantics=("parallel",),
        ),
    )(h)
    del topk_vals

    R3 = min(256, B)
    F3 = min(4096, NF)
    x_hat = pl.pallas_call(
        _dec_kernel,
        grid=(B // R3, NF // F3),
        in_specs=[
            pl.BlockSpec((R3, F3), lambda i, j: (i, j)),
            pl.BlockSpec((D, F3), lambda i, j: (0, j)),
            pl.BlockSpec((1, D), lambda i, j: (0, 0)),
        ],
        out_specs=pl.BlockSpec((R3, D), lambda i, j: (i, 0)),
        out_shape=jax.ShapeDtypeStruct((B, D), jnp.float32),
        compiler_params=pltpu.CompilerParams(
            dimension_semantics=("parallel", "arbitrary"),
        ),
    )(h_sparse, W_dec, b_dec2)

    return (x_hat, h_sparse, topk_idx)


# speedup vs baseline: 5.9637x; 1.0051x over previous
import jax
import jax.numpy as jnp
from jax.experimental import pallas as pl
from jax.experimental.pallas import tpu as pltpu

_NEG_INF = float("-inf")


def _enc_kernel(x_ref, w_ref, b_ref, h_ref):
    h_ref[...] = (
        jax.lax.dot_general(
            x_ref[...], w_ref[...],
            (((1,), (1,)), ((), ())),
            preferred_element_type=jnp.float32,
        )
        + b_ref[...]
    )


def _slow_topk(h, iota, k, hs_ref, vals_ref, idx_ref):
    rows, nf = h.shape
    kiota = jax.lax.broadcasted_iota(jnp.int32, (rows, k), 1)

    def body(i, carry):
        hm, vals, idxs = carry
        m = jnp.max(hm, axis=1, keepdims=True)
        ismax = hm == m
        idx = jnp.min(jnp.where(ismax, iota, nf), axis=1, keepdims=True)
        sel = iota == idx
        slot = kiota == i
        vals = jnp.where(slot, m, vals)
        idxs = jnp.where(slot, idx, idxs)
        return jnp.where(sel, _NEG_INF, hm), vals, idxs

    vals0 = jnp.zeros((rows, k), jnp.float32)
    idxs0 = jnp.zeros((rows, k), jnp.int32)
    hm_final, vals, idxs = jax.lax.fori_loop(
        0, k, body, (h, vals0, idxs0), unroll=False)
    vals_ref[...] = vals
    idx_ref[...] = idxs
    mask = hm_final == _NEG_INF
    hs_ref[...] = jnp.where(mask, jnp.maximum(h, 0.0), 0.0)


def _topk_kernel(h_ref, hs_ref, vals_ref, idx_ref, *, k, s=6, chunk=128):
    h = h_ref[...]
    rows, nf = h.shape
    nper = nf // chunk
    h3 = h.reshape(rows, nper, chunk)
    jio = jax.lax.broadcasted_iota(jnp.int32, (rows, nper, chunk), 1)
    lane2 = jax.lax.broadcasted_iota(jnp.int32, (rows, chunk), 1)
    iota = jax.lax.broadcasted_iota(jnp.int32, (rows, nf), 1)

    cand_v = []
    cand_i = []
    h3m = h3
    for _ in range(s):
        m = jnp.max(h3m, axis=1)
        jidx = jnp.argmax(h3m, axis=1).astype(jnp.int32)
        sel = jio == jidx[:, None, :]
        h3m = jnp.where(sel, _NEG_INF, h3m)
        cand_v.append(m)
        cand_i.append(jidx * chunk + lane2)
    v_last = cand_v[-1]
    cv = jnp.concatenate(cand_v, axis=1)
    ci = jnp.concatenate(cand_i, axis=1)

    kiota = jax.lax.broadcasted_iota(jnp.int32, (rows, k), 1)

    def mbody(i, carry):
        cvm, vals, idxs = carry
        m = jnp.max(cvm, axis=1, keepdims=True)
        oidx = jnp.min(jnp.where(cvm == m, ci, nf), axis=1, keepdims=True)
        selc = ci == oidx
        slot = kiota == i
        vals = jnp.where(slot, m, vals)
        idxs = jnp.where(slot, oidx, idxs)
        return jnp.where(selc, _NEG_INF, cvm), vals, idxs

    vals0 = jnp.zeros((rows, k), jnp.float32)
    idxs0 = jnp.zeros((rows, k), jnp.int32)
    _, vals, idxs = jax.lax.fori_loop(
        0, k, mbody, (cv, vals0, idxs0), unroll=8)

    t_hat = jnp.min(vals, axis=1, keepdims=True)
    viol_cert = jnp.max(v_last, axis=1, keepdims=True) >= t_hat
    ge = (h >= t_hat).astype(jnp.float32)
    cnt = jnp.sum(ge, axis=1, keepdims=True)
    bad_rows = jnp.logical_or(viol_cert, cnt != float(k))
    bad = jnp.max(bad_rows.astype(jnp.int32)) > 0

    @pl.when(jnp.logical_not(bad))
    def _fast():
        vals_ref[...] = vals
        idx_ref[...] = idxs
        hs_ref[...] = jnp.where(h >= t_hat, jnp.maximum(h, 0.0), 0.0)

    @pl.when(bad)
    def _slow():
        _slow_topk(h, iota, k, hs_ref, vals_ref, idx_ref)


def _dec_kernel(hs_ref, w_ref, b_ref, o_ref):
    @pl.when(pl.program_id(1) == 0)
    def _():
        o_ref[...] = jnp.broadcast_to(b_ref[...], o_ref.shape)

    o_ref[...] += jax.lax.dot_general(
        hs_ref[...], w_ref[...],
        (((1,), (1,)), ((), ())),
        preferred_element_type=jnp.float32,
    )


def kernel(x, W_enc, b_enc, W_dec, b_dec):
    B, D = x.shape
    NF = W_enc.shape[0]
    K = 64

    b_enc2 = b_enc.reshape(1, NF)
    b_dec2 = b_dec.reshape(1, D)

    R1 = min(256, B)
    F1 = min(2048, NF)
    h = pl.pallas_call(
        _enc_kernel,
        grid=(B // R1, NF // F1),
        in_specs=[
            pl.BlockSpec((R1, D), lambda i, j: (i, 0)),
            pl.BlockSpec((F1, D), lambda i, j: (j, 0)),
            pl.BlockSpec((1, F1), lambda i, j: (0, j)),
        ],
        out_specs=pl.BlockSpec((R1, F1), lambda i, j: (i, j)),
        out_shape=jax.ShapeDtypeStruct((B, NF), jnp.float32),
        compiler_params=pltpu.CompilerParams(
            dimension_semantics=("parallel", "parallel"),
        ),
    )(x, W_enc, b_enc2)

    R2 = min(32, B)
    import functools
    h_sparse, topk_vals, topk_idx = pl.pallas_call(
        functools.partial(_topk_kernel, k=K),
        grid=(B // R2,),
        in_specs=[pl.BlockSpec((R2, NF), lambda i: (i, 0))],
        out_specs=[
            pl.BlockSpec((R2, NF), lambda i: (i, 0)),
            pl.BlockSpec((R2, K), lambda i: (i, 0)),
            pl.BlockSpec((R2, K), lambda i: (i, 0)),
        ],
        out_shape=[
            jax.ShapeDtypeStruct((B, NF), jnp.float32),
            jax.ShapeDtypeStruct((B, K), jnp.float32),
            jax.ShapeDtypeStruct((B, K), jnp.int32),
        ],
        compiler_params=pltpu.CompilerParams(
            dimension_semantics=("parallel",),
        ),
    )(h)
    del topk_vals

    R3 = min(256, B)
    F3 = min(4096, NF)
    x_hat = pl.pallas_call(
        _dec_kernel,
        grid=(B // R3, NF // F3),
        in_specs=[
            pl.BlockSpec((R3, F3), lambda i, j: (i, j)),
            pl.BlockSpec((D, F3), lambda i, j: (0, j)),
            pl.BlockSpec((1, D), lambda i, j: (0, 0)),
        ],
        out_specs=pl.BlockSpec((R3, D), lambda i, j: (i, 0)),
        out_shape=jax.ShapeDtypeStruct((B, D), jnp.float32),
        compiler_params=pltpu.CompilerParams(
            dimension_semantics=("parallel", "arbitrary"),
        ),
    )(h_sparse, W_dec, b_dec2)

    return (x_hat, h_sparse, topk_idx)
